# Initial kernel scaffold; baseline (speedup 1.0000x reference)
import functools

import jax
import jax.numpy as jnp
from jax import lax
from jax.experimental import pallas as pl
from jax.experimental.pallas import tpu as pltpu
from jax.experimental.pallas import tpu_sc as plsc

N = 10000
D = 128
H = 128
C = 64
E = 320000

NC = 2
NS = 16
NW = NC * NS
CHUNK = 128
CPT = 79
EPAD = NW * CHUNK * CPT
ROWS_PER_TILE = 632
NPAD = NS * ROWS_PER_TILE

_mesh = plsc.VectorSubcoreMesh(core_axis_name="c", subcore_axis_name="s")


@functools.partial(
    pl.kernel,
    out_type=jax.ShapeDtypeStruct((NC, NPAD), jnp.float32),
    mesh=_mesh,
    scratch_types=[
        pltpu.VMEM((CPT, CHUNK), jnp.int32),
        pltpu.VMEM((CHUNK,), jnp.float32),
        pltpu.VMEM_SHARED((NPAD,), jnp.float32),
    ],
)
def _deg_kernel(col_hbm, zeros_hbm, deg_out, col_v, ones_v, deg_sh):
    cid = lax.axis_index("c")
    sid = lax.axis_index("s")
    wid = cid * NS + sid

    base = sid * ROWS_PER_TILE
    pltpu.sync_copy(zeros_hbm.at[pl.ds(base, ROWS_PER_TILE)],
                    deg_sh.at[pl.ds(base, ROWS_PER_TILE)])

    for i in range(CHUNK // 16):
        ones_v[pl.ds(i * 16, 16)] = jnp.ones((16,), jnp.float32)

    pltpu.sync_copy(col_hbm.at[pl.ds(wid * CPT, CPT)], col_v)

    plsc.subcore_barrier()

    def body(j, carry):
        pltpu.sync_copy(ones_v, deg_sh.at[col_v.at[j]], add=True)
        return carry

    lax.fori_loop(0, CPT, body, 0)

    plsc.subcore_barrier()

    pltpu.sync_copy(deg_sh.at[pl.ds(base, ROWS_PER_TILE)],
                    deg_out.at[cid, pl.ds(base, ROWS_PER_TILE)])


def _xs_body(deg_ref, x_ref, xs_ref):
    degsum = deg_ref[:, 0:1] + deg_ref[:, 1:2] + 1.0
    xs_ref[...] = x_ref[...] * lax.rsqrt(degsum)


def _xs_call(degT, x):
    blk = 1000
    return pl.pallas_call(
        _xs_body,
        grid=(N // blk,),
        in_specs=[
            pl.BlockSpec((blk, 2), lambda i: (i, 0)),
            pl.BlockSpec((blk, D), lambda i: (i, 0)),
        ],
        out_specs=pl.BlockSpec((blk, D), lambda i: (i, 0)),
        out_shape=jax.ShapeDtypeStruct((N, D), jnp.float32),
    )(degT, x)


@functools.partial(
    pl.kernel,
    out_type=jax.ShapeDtypeStruct((NC, NPAD, D), jnp.float32),
    mesh=_mesh,
    scratch_types=[
        pltpu.VMEM((CPT, CHUNK), jnp.int32),
        pltpu.VMEM((CPT, CHUNK), jnp.int32),
        pltpu.VMEM((CHUNK, D), jnp.float32),
        pltpu.VMEM_SHARED((NPAD, D), jnp.float32),
    ],
)
def _scatter_kernel(xs_hbm, row_hbm, col_hbm, zeros_hbm, t_out,
                    row_v, col_v, buf, t_sh):
    cid = lax.axis_index("c")
    sid = lax.axis_index("s")
    wid = cid * NS + sid

    base = sid * ROWS_PER_TILE
    pltpu.sync_copy(zeros_hbm.at[pl.ds(base, ROWS_PER_TILE)],
                    t_sh.at[pl.ds(base, ROWS_PER_TILE)])

    pltpu.sync_copy(row_hbm.at[pl.ds(wid * CPT, CPT)], row_v)
    pltpu.sync_copy(col_hbm.at[pl.ds(wid * CPT, CPT)], col_v)

    plsc.subcore_barrier()

    def body(j, carry):
        pltpu.sync_copy(xs_hbm.at[row_v.at[j]], buf)
        pltpu.sync_copy(buf, t_sh.at[col_v.at[j]], add=True)
        return carry

    lax.fori_loop(0, CPT, body, 0)

    plsc.subcore_barrier()

    pltpu.sync_copy(t_sh.at[pl.ds(base, ROWS_PER_TILE)],
                    t_out.at[cid, pl.ds(base, ROWS_PER_TILE)])


def _elu(v):
    return jnp.maximum(v, 0.0) + jnp.expm1(jnp.minimum(v, 0.0))


def _dense_body(t_ref, deg_ref, xs_ref, w0_ref, b0_ref, l0_ref, lb0_ref,
                w1_ref, b1_ref, l1_ref, lb1_ref, owa_ref, owb_ref, ob_ref,
                out_ref):
    dis = lax.rsqrt(deg_ref[:, 0:1] + deg_ref[:, 1:2] + 1.0)
    agg = (t_ref[0] + t_ref[1] + xs_ref[...]) * dis
    f32 = jnp.float32
    a0 = _elu(jnp.dot(agg, w0_ref[...], preferred_element_type=f32) + b0_ref[...])
    y0 = jnp.dot(a0, l0_ref[...], preferred_element_type=f32) + lb0_ref[...]
    a1 = _elu(jnp.dot(agg, w1_ref[...], preferred_element_type=f32) + b1_ref[...])
    y1 = jnp.dot(a1, l1_ref[...], preferred_element_type=f32) + lb1_ref[...]
    out_ref[...] = (jnp.dot(y0, owa_ref[...], preferred_element_type=f32)
                    + jnp.dot(y1, owb_ref[...], preferred_element_type=f32)
                    + ob_ref[...])


def _dense_call(t2, degT, xs, w0, b0, l0, lb0, w1, b1, l1, lb1, owa, owb, ob):
    blk = 1000

    def full(shape):
        return pl.BlockSpec(shape, lambda i, _s=shape: tuple(0 for _ in _s))

    return pl.pallas_call(
        _dense_body,
        grid=(N // blk,),
        in_specs=[
            pl.BlockSpec((NC, blk, D), lambda i: (0, i, 0)),
            pl.BlockSpec((blk, 2), lambda i: (i, 0)),
            pl.BlockSpec((blk, D), lambda i: (i, 0)),
            full((D, H)), full((1, H)), full((H, H)), full((1, H)),
            full((D, H)), full((1, H)), full((H, H)), full((1, H)),
            full((H, C)), full((H, C)), full((1, C)),
        ],
        out_specs=pl.BlockSpec((blk, C), lambda i: (i, 0)),
        out_shape=jax.ShapeDtypeStruct((N, C), jnp.float32),
    )(t2, degT, xs, w0, b0, l0, lb0, w1, b1, l1, lb1, owa, owb, ob)


def kernel(x, edge_index, conv0_W, conv0_b, lin0_W, lin0_b,
           conv1_W, conv1_b, lin1_W, lin1_b, out_W, out_b):
    pad = EPAD - E
    row_p = jnp.concatenate(
        [edge_index[0], jnp.zeros((pad,), jnp.int32)]).reshape(EPAD // CHUNK, CHUNK)
    col_p = jnp.concatenate(
        [edge_index[1], jnp.full((pad,), N, jnp.int32)]).reshape(EPAD // CHUNK, CHUNK)

    deg2 = _deg_kernel(col_p, jnp.zeros((NPAD,), jnp.float32))
    degT = deg2.T

    xs = _xs_call(degT[:N], x)

    t2 = _scatter_kernel(xs, row_p, col_p, jnp.zeros((NPAD, D), jnp.float32))

    return _dense_call(
        t2[:, :N], degT[:N], xs,
        conv0_W, conv0_b.reshape(1, H), lin0_W, lin0_b.reshape(1, H),
        conv1_W, conv1_b.reshape(1, H), lin1_W, lin1_b.reshape(1, H),
        out_W[:H], out_W[H:], out_b.reshape(1, C))


# speedup vs baseline: 18.3242x; 18.3242x over previous
import functools

import jax
import jax.numpy as jnp
from jax import lax
from jax.experimental import pallas as pl
from jax.experimental.pallas import tpu as pltpu
from jax.experimental.pallas import tpu_sc as plsc

N = 10000
D = 128
H = 128
C = 64
E = 320000

NC = 2
NS = 16
NW = NC * NS
CHUNK = 128
CPT = 80
EPAD = NW * CHUNK * CPT
ROWS_PER_TILE = 640
NPAD = NS * ROWS_PER_TILE

_mesh = plsc.VectorSubcoreMesh(core_axis_name="c", subcore_axis_name="s")


@functools.partial(
    pl.kernel,
    out_type=jax.ShapeDtypeStruct((NC * NPAD,), jnp.float32),
    mesh=_mesh,
    scratch_types=[
        pltpu.VMEM((CPT, CHUNK), jnp.int32),
        pltpu.VMEM((CHUNK,), jnp.float32),
        pltpu.VMEM_SHARED((NPAD,), jnp.float32),
    ],
)
def _deg_kernel(col_hbm, zeros_hbm, deg_out, col_v, ones_v, deg_sh):
    cid = lax.axis_index("c")
    sid = lax.axis_index("s")
    wid = cid * NS + sid

    base = sid * ROWS_PER_TILE
    pltpu.sync_copy(zeros_hbm.at[pl.ds(base, ROWS_PER_TILE)],
                    deg_sh.at[pl.ds(base, ROWS_PER_TILE)])

    for i in range(CHUNK // 16):
        ones_v[pl.ds(i * 16, 16)] = jnp.ones((16,), jnp.float32)

    pltpu.sync_copy(col_hbm.at[pl.ds(wid * CPT, CPT)], col_v)

    plsc.subcore_barrier()

    def body(j, carry):
        pltpu.sync_copy(ones_v, deg_sh.at[col_v.at[j]], add=True)
        return carry

    lax.fori_loop(0, CPT, body, 0)

    plsc.subcore_barrier()

    pltpu.sync_copy(deg_sh.at[pl.ds(base, ROWS_PER_TILE)],
                    deg_out.at[pl.ds(cid * NPAD + base, ROWS_PER_TILE)])


def _xs_body(deg_ref, x_ref, xs_ref):
    degsum = deg_ref[:, 0:1] + deg_ref[:, 1:2] + 1.0
    xs_ref[...] = x_ref[...] * lax.rsqrt(degsum)


def _xs_call(degT, x):
    blk = 1000
    return pl.pallas_call(
        _xs_body,
        grid=(N // blk,),
        in_specs=[
            pl.BlockSpec((blk, 2), lambda i: (i, 0)),
            pl.BlockSpec((blk, D), lambda i: (i, 0)),
        ],
        out_specs=pl.BlockSpec((blk, D), lambda i: (i, 0)),
        out_shape=jax.ShapeDtypeStruct((N, D), jnp.float32),
    )(degT, x)


@functools.partial(
    pl.kernel,
    out_type=jax.ShapeDtypeStruct((NC, NPAD, D), jnp.float32),
    mesh=_mesh,
    scratch_types=[
        pltpu.VMEM((CPT, CHUNK), jnp.int32),
        pltpu.VMEM((CPT, CHUNK), jnp.int32),
        pltpu.VMEM((CHUNK, D), jnp.float32),
        pltpu.VMEM_SHARED((NPAD, D), jnp.float32),
    ],
)
def _scatter_kernel(xs_hbm, row_hbm, col_hbm, zeros_hbm, t_out,
                    row_v, col_v, buf, t_sh):
    cid = lax.axis_index("c")
    sid = lax.axis_index("s")
    wid = cid * NS + sid

    base = sid * ROWS_PER_TILE
    pltpu.sync_copy(zeros_hbm.at[pl.ds(base, ROWS_PER_TILE)],
                    t_sh.at[pl.ds(base, ROWS_PER_TILE)])

    pltpu.sync_copy(row_hbm.at[pl.ds(wid * CPT, CPT)], row_v)
    pltpu.sync_copy(col_hbm.at[pl.ds(wid * CPT, CPT)], col_v)

    plsc.subcore_barrier()

    def body(j, carry):
        pltpu.sync_copy(xs_hbm.at[row_v.at[j]], buf)
        pltpu.sync_copy(buf, t_sh.at[col_v.at[j]], add=True)
        return carry

    lax.fori_loop(0, CPT, body, 0)

    plsc.subcore_barrier()

    pltpu.sync_copy(t_sh.at[pl.ds(base, ROWS_PER_TILE)],
                    t_out.at[cid, pl.ds(base, ROWS_PER_TILE)])


def _elu(v):
    return jnp.maximum(v, 0.0) + (jnp.exp(jnp.minimum(v, 0.0)) - 1.0)


def _dense_body(t_ref, deg_ref, xs_ref, w0_ref, b0_ref, l0_ref, lb0_ref,
                w1_ref, b1_ref, l1_ref, lb1_ref, owa_ref, owb_ref, ob_ref,
                out_ref):
    dis = lax.rsqrt(deg_ref[:, 0:1] + deg_ref[:, 1:2] + 1.0)
    agg = (t_ref[0] + t_ref[1] + xs_ref[...]) * dis
    f32 = jnp.float32
    a0 = _elu(jnp.dot(agg, w0_ref[...], preferred_element_type=f32) + b0_ref[...])
    y0 = jnp.dot(a0, l0_ref[...], preferred_element_type=f32) + lb0_ref[...]
    a1 = _elu(jnp.dot(agg, w1_ref[...], preferred_element_type=f32) + b1_ref[...])
    y1 = jnp.dot(a1, l1_ref[...], preferred_element_type=f32) + lb1_ref[...]
    out_ref[...] = (jnp.dot(y0, owa_ref[...], preferred_element_type=f32)
                    + jnp.dot(y1, owb_ref[...], preferred_element_type=f32)
                    + ob_ref[...])


def _dense_call(t2, degT, xs, w0, b0, l0, lb0, w1, b1, l1, lb1, owa, owb, ob):
    blk = 1000

    def full(shape):
        return pl.BlockSpec(shape, lambda i, _s=shape: tuple(0 for _ in _s))

    return pl.pallas_call(
        _dense_body,
        grid=(N // blk,),
        in_specs=[
            pl.BlockSpec((NC, blk, D), lambda i: (0, i, 0)),
            pl.BlockSpec((blk, 2), lambda i: (i, 0)),
            pl.BlockSpec((blk, D), lambda i: (i, 0)),
            full((D, H)), full((1, H)), full((H, H)), full((1, H)),
            full((D, H)), full((1, H)), full((H, H)), full((1, H)),
            full((H, C)), full((H, C)), full((1, C)),
        ],
        out_specs=pl.BlockSpec((blk, C), lambda i: (i, 0)),
        out_shape=jax.ShapeDtypeStruct((N, C), jnp.float32),
    )(t2, degT, xs, w0, b0, l0, lb0, w1, b1, l1, lb1, owa, owb, ob)


def kernel(x, edge_index, conv0_W, conv0_b, lin0_W, lin0_b,
           conv1_W, conv1_b, lin1_W, lin1_b, out_W, out_b):
    pad = EPAD - E
    row_p = jnp.concatenate(
        [edge_index[0], jnp.zeros((pad,), jnp.int32)]).reshape(EPAD // CHUNK, CHUNK)
    col_p = jnp.concatenate(
        [edge_index[1], jnp.full((pad,), N, jnp.int32)]).reshape(EPAD // CHUNK, CHUNK)

    deg2 = _deg_kernel(col_p, jnp.zeros((NPAD,), jnp.float32)).reshape(NC, NPAD)
    degT = deg2.T

    xs = _xs_call(degT[:N], x)

    t2 = _scatter_kernel(xs, row_p, col_p, jnp.zeros((NPAD, D), jnp.float32))

    return _dense_call(
        t2[:, :N], degT[:N], xs,
        conv0_W, conv0_b.reshape(1, H), lin0_W, lin0_b.reshape(1, H),
        conv1_W, conv1_b.reshape(1, H), lin1_W, lin1_b.reshape(1, H),
        out_W[:H], out_W[H:], out_b.reshape(1, C))
